# baseline (speedup 1.0000x reference)
import functools

import jax
import jax.numpy as jnp
from jax import lax
from jax.experimental import pallas as pl
from jax.experimental.pallas import tpu as pltpu
from jax.experimental.pallas import tpu_sc as plsc

NUM_USERS = 1000000
EMBED_DIM = 64
BATCH = 16384
_CHUNK = 32
_L = 16


@jax.jit
def _embed_lookup(user_inputs, table):
    info = plsc.get_sparse_core_info()
    nw = info.num_cores * info.num_subcores
    b_per_w = BATCH // nw
    n_chunks = b_per_w // _CHUNK
    mesh = plsc.VectorSubcoreMesh(core_axis_name="c", subcore_axis_name="s")

    idx = user_inputs.astype(jnp.int32)
    tbl3 = table.reshape(NUM_USERS // 8, 8, EMBED_DIM)

    @functools.partial(
        pl.kernel,
        mesh=mesh,
        out_type=jax.ShapeDtypeStruct((BATCH // 8, 8, EMBED_DIM), jnp.float32),
        scratch_types=[
            pltpu.VMEM((b_per_w,), jnp.int32),
            pltpu.VMEM((_CHUNK, 8, EMBED_DIM), jnp.float32),
            pltpu.VMEM((_CHUNK, 8, EMBED_DIM), jnp.float32),
            pltpu.VMEM((_CHUNK // 8, 8, EMBED_DIM), jnp.float32),
            pltpu.VMEM((_CHUNK // 8, 8, EMBED_DIM), jnp.float32),
            pltpu.SemaphoreType.DMA,
            pltpu.SemaphoreType.DMA,
            pltpu.SemaphoreType.DMA,
        ],
    )
    def gather_rows(idx_hbm, tbl_hbm, out_hbm, idx_v, blks_a, blks_b,
                    rowsc_a, rowsc_b, sem_a, sem_b, wsem):
        wid = lax.axis_index("s") * info.num_cores + lax.axis_index("c")
        base = wid * b_per_w
        pltpu.sync_copy(idx_hbm.at[pl.ds(base, b_per_w)], idx_v)
        bufs = (blks_a, blks_b)
        sems = (sem_a, sem_b)
        rowsc = (rowsc_a, rowsc_b)

        def fire(c, buf, sem):
            @plsc.parallel_loop(0, _CHUNK // _L)
            def body(grp):
                v = idx_v[pl.ds(c * _CHUNK + grp * _L, _L)]
                for lane in range(_L):
                    j = v[lane] >> 3
                    pltpu.async_copy(
                        tbl_hbm.at[j],
                        buf.at[grp * _L + lane],
                        sem,
                    )

        def drain_select(c, buf, sem, rc):
            pltpu.make_async_copy(
                tbl_hbm.at[pl.ds(0, _CHUNK)], buf, sem
            ).wait()

            @plsc.parallel_loop(0, _CHUNK // _L)
            def body(grp):
                v = idx_v[pl.ds(c * _CHUNK + grp * _L, _L)]
                for lane in range(_L):
                    i = grp * _L + lane
                    s = v[lane] & 7
                    for q in range(EMBED_DIM // _L):
                        rc[i >> 3, i & 7, pl.ds(_L * q, _L)] = buf[
                            i, s, pl.ds(_L * q, _L)
                        ]

        fire(0, bufs[0], sems[0])
        for c in range(n_chunks):
            if c + 1 < n_chunks:
                fire(c + 1, bufs[(c + 1) % 2], sems[(c + 1) % 2])
            if c >= 2:
                pltpu.make_async_copy(
                    tbl_hbm.at[pl.ds(0, _CHUNK // 8)], rowsc[c % 2], wsem
                ).wait()
            drain_select(c, bufs[c % 2], sems[c % 2], rowsc[c % 2])
            pltpu.async_copy(
                rowsc[c % 2],
                out_hbm.at[pl.ds(wid * (b_per_w // 8) + c * (_CHUNK // 8),
                                 _CHUNK // 8)],
                wsem,
            )
        for c in (n_chunks - 2, n_chunks - 1):
            pltpu.make_async_copy(
                tbl_hbm.at[pl.ds(0, _CHUNK // 8)], rowsc[c % 2], wsem
            ).wait()

    out3 = gather_rows(idx, tbl3)
    return out3.reshape(BATCH, EMBED_DIM)


def kernel(user_inputs, table):
    return _embed_lookup(user_inputs, table)


# speedup vs baseline: 1.0305x; 1.0305x over previous
import functools

import jax
import jax.numpy as jnp
from jax import lax
from jax.experimental import pallas as pl
from jax.experimental.pallas import tpu as pltpu
from jax.experimental.pallas import tpu_sc as plsc

NUM_USERS = 1000000
EMBED_DIM = 64
BATCH = 16384
_CHUNK = 32
_L = 16


@jax.jit
def _embed_lookup(user_inputs, table):
    info = plsc.get_sparse_core_info()
    nw = info.num_cores * info.num_subcores
    b_per_w = BATCH // nw
    n_chunks = b_per_w // _CHUNK
    mesh = plsc.VectorSubcoreMesh(core_axis_name="c", subcore_axis_name="s")

    idx = user_inputs.astype(jnp.int32)
    tbl3 = table.reshape(NUM_USERS // 8, 8, EMBED_DIM)

    @functools.partial(
        pl.kernel,
        mesh=mesh,
        out_type=jax.ShapeDtypeStruct((BATCH // 8, 8, EMBED_DIM), jnp.float32),
        scratch_types=[
            pltpu.VMEM((b_per_w,), jnp.int32),
            pltpu.VMEM((_CHUNK, 8, EMBED_DIM), jnp.float32),
            pltpu.VMEM((_CHUNK, 8, EMBED_DIM), jnp.float32),
            pltpu.VMEM((_CHUNK // 8, 8, EMBED_DIM), jnp.float32),
            pltpu.VMEM((_CHUNK // 8, 8, EMBED_DIM), jnp.float32),
            pltpu.SemaphoreType.DMA,
            pltpu.SemaphoreType.DMA,
            pltpu.SemaphoreType.DMA,
        ],
    )
    def gather_rows(idx_hbm, tbl_hbm, out_hbm, idx_v, blks_a, blks_b,
                    rowsc_a, rowsc_b, sem_a, sem_b, wsem):
        wid = lax.axis_index("s") * info.num_cores + lax.axis_index("c")
        base = wid * b_per_w
        pltpu.sync_copy(idx_hbm.at[pl.ds(base, b_per_w)], idx_v)
        bufs = (blks_a, blks_b)
        sems = (sem_a, sem_b)
        rowsc = (rowsc_a, rowsc_b)

        def fire(c, buf, sem):
            @plsc.parallel_loop(0, _CHUNK // _L)
            def body(grp):
                v = idx_v[pl.ds(c * _CHUNK + grp * _L, _L)]
                for lane in range(_L):
                    j = v[lane] >> 3
                    pltpu.async_copy(
                        tbl_hbm.at[j],
                        buf.at[grp * _L + lane],
                        sem,
                    )

        def drain_select(c, buf, sem, rc):
            pltpu.make_async_copy(
                tbl_hbm.at[pl.ds(0, _CHUNK)], buf, sem
            ).wait()

            @plsc.parallel_loop(0, _CHUNK // _L)
            def body(grp):
                v = idx_v[pl.ds(c * _CHUNK + grp * _L, _L)]
                for lane in range(_L):
                    i = grp * _L + lane
                    s = v[lane] & 7
                    for q in range(EMBED_DIM // _L):
                        rc[i >> 3, i & 7, pl.ds(_L * q, _L)] = buf[
                            i, s, pl.ds(_L * q, _L)
                        ]

        def reclaim(rc):
            pltpu.make_async_copy(
                tbl_hbm.at[pl.ds(0, _CHUNK // 8)], rc, wsem
            ).wait()

        def write_out(c, rc):
            pltpu.async_copy(
                rc,
                out_hbm.at[pl.ds(wid * (b_per_w // 8) + c * (_CHUNK // 8),
                                 _CHUNK // 8)],
                wsem,
            )

        fire(0, bufs[0], sems[0])

        def pair(p, carry):
            c0 = p * 2
            fire(c0 + 1, bufs[1], sems[1])
            pl.when(p >= 1)(lambda: reclaim(rowsc[0]))
            drain_select(c0, bufs[0], sems[0], rowsc[0])
            write_out(c0, rowsc[0])
            pl.when(p < n_chunks // 2 - 1)(
                lambda: fire(c0 + 2, bufs[0], sems[0])
            )
            pl.when(p >= 1)(lambda: reclaim(rowsc[1]))
            drain_select(c0 + 1, bufs[1], sems[1], rowsc[1])
            write_out(c0 + 1, rowsc[1])
            return carry

        lax.fori_loop(0, n_chunks // 2, pair, 0, unroll=False)
        reclaim(rowsc[0])
        reclaim(rowsc[1])

    out3 = gather_rows(idx, tbl3)
    return out3.reshape(BATCH, EMBED_DIM)


def kernel(user_inputs, table):
    return _embed_lookup(user_inputs, table)


# speedup vs baseline: 1.0319x; 1.0014x over previous
import functools

import jax
import jax.numpy as jnp
from jax import lax
from jax.experimental import pallas as pl
from jax.experimental.pallas import tpu as pltpu
from jax.experimental.pallas import tpu_sc as plsc

NUM_USERS = 1000000
EMBED_DIM = 64
BATCH = 16384
_CHUNK = 16
_L = 16


@jax.jit
def _embed_lookup(user_inputs, table):
    info = plsc.get_sparse_core_info()
    nw = info.num_cores * info.num_subcores
    b_per_w = BATCH // nw
    n_chunks = b_per_w // _CHUNK
    mesh = plsc.VectorSubcoreMesh(core_axis_name="c", subcore_axis_name="s")

    idx = user_inputs.astype(jnp.int32)
    tbl3 = table.reshape(NUM_USERS // 8, 8, EMBED_DIM)

    @functools.partial(
        pl.kernel,
        mesh=mesh,
        out_type=jax.ShapeDtypeStruct((BATCH // 8, 8, EMBED_DIM), jnp.float32),
        scratch_types=[
            pltpu.VMEM((b_per_w,), jnp.int32),
            pltpu.VMEM((_CHUNK, 8, EMBED_DIM), jnp.float32),
            pltpu.VMEM((_CHUNK, 8, EMBED_DIM), jnp.float32),
            pltpu.VMEM((_CHUNK, 8, EMBED_DIM), jnp.float32),
            pltpu.VMEM((_CHUNK, 8, EMBED_DIM), jnp.float32),
            pltpu.VMEM((_CHUNK // 8, 8, EMBED_DIM), jnp.float32),
            pltpu.VMEM((_CHUNK // 8, 8, EMBED_DIM), jnp.float32),
            pltpu.SemaphoreType.DMA,
            pltpu.SemaphoreType.DMA,
            pltpu.SemaphoreType.DMA,
            pltpu.SemaphoreType.DMA,
            pltpu.SemaphoreType.DMA,
        ],
    )
    def gather_rows(idx_hbm, tbl_hbm, out_hbm, idx_v, blks_a, blks_b,
                    blks_c, blks_d, rowsc_a, rowsc_b, sem_a, sem_b,
                    sem_c, sem_d, wsem):
        wid = lax.axis_index("s") * info.num_cores + lax.axis_index("c")
        base = wid * b_per_w
        pltpu.sync_copy(idx_hbm.at[pl.ds(base, b_per_w)], idx_v)
        bufs = (blks_a, blks_b, blks_c, blks_d)
        sems = (sem_a, sem_b, sem_c, sem_d)
        rowsc = (rowsc_a, rowsc_b)

        def fire(c, buf, sem):
            @plsc.parallel_loop(0, _CHUNK // _L)
            def body(grp):
                v = idx_v[pl.ds(c * _CHUNK + grp * _L, _L)]
                for lane in range(_L):
                    j = v[lane] >> 3
                    pltpu.async_copy(
                        tbl_hbm.at[j],
                        buf.at[grp * _L + lane],
                        sem,
                    )

        def drain_select(c, buf, sem, rc):
            pltpu.make_async_copy(
                tbl_hbm.at[pl.ds(0, _CHUNK)], buf, sem
            ).wait()

            @plsc.parallel_loop(0, _CHUNK // _L)
            def body(grp):
                v = idx_v[pl.ds(c * _CHUNK + grp * _L, _L)]
                for lane in range(_L):
                    i = grp * _L + lane
                    s = v[lane] & 7
                    for q in range(EMBED_DIM // _L):
                        rc[i >> 3, i & 7, pl.ds(_L * q, _L)] = buf[
                            i, s, pl.ds(_L * q, _L)
                        ]

        def reclaim(rc):
            pltpu.make_async_copy(
                tbl_hbm.at[pl.ds(0, _CHUNK // 8)], rc, wsem
            ).wait()

        def write_out(c, rc):
            pltpu.async_copy(
                rc,
                out_hbm.at[pl.ds(wid * (b_per_w // 8) + c * (_CHUNK // 8),
                                 _CHUNK // 8)],
                wsem,
            )

        fire(0, bufs[0], sems[0])
        fire(1, bufs[1], sems[1])

        def quad(p, carry):
            c0 = p * 4
            for k in range(4):
                nxt = c0 + k + 2
                pl.when(nxt < n_chunks)(
                    lambda nxt=nxt, k=k: fire(
                        nxt, bufs[(k + 2) % 4], sems[(k + 2) % 4]
                    )
                )
                pl.when(c0 + k >= 2)(
                    lambda k=k: reclaim(rowsc[k % 2])
                )
                drain_select(c0 + k, bufs[k], sems[k], rowsc[k % 2])
                write_out(c0 + k, rowsc[k % 2])
            return carry

        lax.fori_loop(0, n_chunks // 4, quad, 0, unroll=False)
        reclaim(rowsc[0])
        reclaim(rowsc[1])

    out3 = gather_rows(idx, tbl3)
    return out3.reshape(BATCH, EMBED_DIM)


def kernel(user_inputs, table):
    return _embed_lookup(user_inputs, table)
